# baseline (speedup 1.0000x reference)
import functools

import jax
import jax.numpy as jnp
import numpy as np
from jax import lax
from jax.experimental import pallas as pl
from jax.experimental.pallas import tpu as pltpu
from jax.experimental.pallas import tpu_sc as plsc

H = 512
W = 512
PSZ = 16
CCH = 3
NB = 32
WW = W // PSZ
NR = H // PSZ
ROWS48 = CCH * PSZ
DPATCH = PSZ * PSZ * CCH

CH = 32
RING = 3
CHB = CH * DPATCH

def _vextract(ref, i):
    base = (i >> 4) << 4
    v = ref[pl.ds(base, PSZ)]
    lane = i - base
    sel = jnp.where(lax.iota(jnp.int32, PSZ) == lane, v, 0)
    return jnp.sum(sel)


@functools.lru_cache(maxsize=None)
def _make_sc_call(K: int, Kp: int):
    nblk, rem = divmod(K, CH)
    mesh = plsc.VectorSubcoreMesh(core_axis_name="c", subcore_axis_name="s")

    @functools.partial(
        pl.kernel,
        mesh=mesh,
        compiler_params=pltpu.CompilerParams(
            needs_layout_passes=False, use_tc_tiling_on_sc=True),
        out_type=jax.ShapeDtypeStruct((NB, K, DPATCH), jnp.float32),
        scratch_types=[
            pltpu.VMEM((ROWS48, W), jnp.float32),
            pltpu.VMEM((ROWS48, W), jnp.float32),
            pltpu.VMEM((RING * CH, DPATCH), jnp.float32),
            pltpu.VMEM((Kp,), jnp.int32),
            pltpu.VMEM((ROWS48,), jnp.int32),
            [pltpu.SemaphoreType.DMA for _ in range(4)],
        ],
    )
    def sc_kernel(img2d, qcol, starts, out, strip_a, strip_b, ring,
                  qcol_v, starts_v, sems):
        b = lax.axis_index("s") * 2 + lax.axis_index("c")
        sem_sa, sem_sb, sem_wb, sem_rem = sems
        pltpu.sync_copy(qcol, qcol_v)
        pltpu.sync_copy(starts, starts_v)

        def fire_strip(r, strip, sem):
            return [
                pltpu.async_copy(
                    img2d.at[pl.ds((b * CCH + c) * H + PSZ * r, PSZ), :],
                    strip.at[pl.ds(c * PSZ, PSZ), :], sem)
                for c in range(CCH)
            ]

        def drain_strip(strip, sem):
            for c in range(CCH):
                pltpu.make_async_copy(
                    img2d.at[pl.ds(0, PSZ), :],
                    strip.at[pl.ds(c * PSZ, PSZ), :], sem).wait()

        def process(r, strip):
            s0 = _vextract(starts_v, r)
            s1 = _vextract(starts_v, r + 1)

            @plsc.parallel_loop(s0, s1, unroll=2)
            def pbody(k):
                cb = _vextract(qcol_v, k)
                km = k - (k // (RING * CH)) * (RING * CH)
                rowv = jnp.full((PSZ,), km, jnp.int32)
                iota3 = lax.iota(jnp.int32, PSZ) * CCH
                for j in range(ROWS48):
                    ipj = iota3 + ((j % PSZ) * ROWS48 + j // PSZ)
                    vec = strip[j, pl.ds(cb, PSZ)]
                    plsc.store_scatter(ring, [rowv, ipj], vec)

            def fbody(blk, carry):
                @pl.when(blk >= RING - 2)
                def _():
                    pltpu.make_async_copy(
                        out.at[0, pl.ds(0, CH), :], ring.at[pl.ds(0, CH), :],
                        sem_wb).wait()
                slot = blk - (blk // RING) * RING
                pltpu.async_copy(
                    ring.at[pl.ds(slot * CH, CH), :],
                    out.at[b, pl.ds(blk * CH, CH), :], sem_wb)
                return carry

            lax.fori_loop(s0 // CH, s1 // CH, fbody, 0)

        h0 = fire_strip(0, strip_a, sem_sa)
        del h0

        def srbody(rr, carry):
            r0 = rr * 2
            hb = fire_strip(r0 + 1, strip_b, sem_sb)
            drain_strip(strip_a, sem_sa)
            process(r0, strip_a)

            @pl.when(r0 + 2 < NR)
            def _():
                fire_strip(r0 + 2, strip_a, sem_sa)

            for h in hb:
                h.wait()
            process(r0 + 1, strip_b)
            return carry

        lax.fori_loop(0, NR // 2, srbody, 0)

        if rem:
            slot = nblk - (nblk // RING) * RING
            pltpu.async_copy(
                ring.at[pl.ds(slot * CH, rem), :],
                out.at[b, pl.ds(nblk * CH, rem), :], sem_rem)
        for _ in range(nblk - max(0, nblk - (RING - 2))):
            pltpu.make_async_copy(
                out.at[0, pl.ds(0, CH), :], ring.at[pl.ds(0, CH), :],
                sem_wb).wait()
        if rem:
            pltpu.make_async_copy(
                out.at[0, pl.ds(0, rem), :], ring.at[pl.ds(0, rem), :],
                sem_rem).wait()

    return sc_kernel


def kernel(img, patch_indices):
    K = patch_indices.shape[0]
    Kp = ((K + PSZ) // PSZ) * PSZ
    img2d = img.reshape(NB * CCH * H, W)
    q = (patch_indices % WW).astype(jnp.int32)
    qcol = jnp.zeros((Kp,), jnp.int32).at[:K].set(q * PSZ)
    r = (patch_indices // WW).astype(jnp.int32)
    starts = jnp.sum(r[None, :] < jnp.arange(ROWS48, dtype=jnp.int32)[:, None],
                     axis=1, dtype=jnp.int32)
    return _make_sc_call(K, Kp)(img2d, qcol, starts)


# speedup vs baseline: 1.7641x; 1.7641x over previous
import functools

import jax
import jax.numpy as jnp
import numpy as np
from jax import lax
from jax.experimental import pallas as pl
from jax.experimental.pallas import tpu as pltpu
from jax.experimental.pallas import tpu_sc as plsc

H = 512
W = 512
PSZ = 16
CCH = 3
NB = 32
WW = W // PSZ
NR = H // PSZ
ROWS48 = CCH * PSZ
DPATCH = PSZ * PSZ * CCH

CH = 32
RING = 3
CHB = CH * DPATCH

def _vextract(ref, i, nmax=None):
    base = (i >> 4) << 4
    if nmax is not None:
        base = jnp.minimum(base, nmax - PSZ)
    v = ref[pl.ds(base, PSZ)]
    lane = i - base
    sel = jnp.where(lax.iota(jnp.int32, PSZ) == lane, v, 0)
    return jnp.sum(sel)


@functools.lru_cache(maxsize=None)
def _make_sc_call(K: int, Kp: int):
    nblk, rem = divmod(K, CH)
    mesh = plsc.VectorSubcoreMesh(core_axis_name="c", subcore_axis_name="s")

    @functools.partial(
        pl.kernel,
        mesh=mesh,
        compiler_params=pltpu.CompilerParams(
            needs_layout_passes=False, use_tc_tiling_on_sc=True),
        out_type=jax.ShapeDtypeStruct((NB, K, DPATCH), jnp.float32),
        scratch_types=[
            pltpu.VMEM((ROWS48, W), jnp.float32),
            pltpu.VMEM((ROWS48, W), jnp.float32),
            pltpu.VMEM((RING * CH, DPATCH), jnp.float32),
            pltpu.VMEM((Kp,), jnp.int32),
            pltpu.VMEM((ROWS48,), jnp.int32),
            [pltpu.SemaphoreType.DMA for _ in range(4)],
        ],
    )
    def sc_kernel(img2d, qcol, starts, out, strip_a, strip_b, ring,
                  qcol_v, starts_v, sems):
        b = lax.axis_index("s") * 2 + lax.axis_index("c")
        sem_sa, sem_sb, sem_wb, sem_rem = sems
        pltpu.sync_copy(qcol, qcol_v)
        pltpu.sync_copy(starts, starts_v)

        def fire_strip(r, strip, sem):
            return [
                pltpu.async_copy(
                    img2d.at[pl.ds((b * CCH + c) * H + PSZ * r, PSZ), :],
                    strip.at[pl.ds(c * PSZ, PSZ), :], sem)
                for c in range(CCH)
            ]

        def drain_strip(strip, sem):
            for c in range(CCH):
                pltpu.make_async_copy(
                    img2d.at[pl.ds(0, PSZ), :],
                    strip.at[pl.ds(c * PSZ, PSZ), :], sem).wait()

        def process(r, strip):
            s0 = _vextract(starts_v, r)
            s1 = _vextract(starts_v, r + 1)

            @plsc.parallel_loop(s0, s1)
            def pbody(k):
                cb = _vextract(qcol_v, k, nmax=Kp)
                km = k - (k // (RING * CH)) * (RING * CH)
                rowv = jnp.full((PSZ,), km, jnp.int32)
                iota3 = lax.iota(jnp.int32, PSZ) * CCH
                for j in range(ROWS48):
                    ipj = iota3 + ((j % PSZ) * ROWS48 + j // PSZ)
                    vec = strip[j, pl.ds(cb, PSZ)]
                    plsc.store_scatter(ring, [rowv, ipj], vec)

            def fbody(blk, carry):
                @pl.when(blk >= RING - 2)
                def _():
                    pltpu.make_async_copy(
                        out.at[0, pl.ds(0, CH), :], ring.at[pl.ds(0, CH), :],
                        sem_wb).wait()
                slot = blk - (blk // RING) * RING
                pltpu.async_copy(
                    ring.at[pl.ds(slot * CH, CH), :],
                    out.at[b, pl.ds(blk * CH, CH), :], sem_wb)
                return carry

            lax.fori_loop(s0 // CH, s1 // CH, fbody, 0)

        h0 = fire_strip(0, strip_a, sem_sa)
        del h0

        def srbody(rr, carry):
            r0 = rr * 2
            hb = fire_strip(r0 + 1, strip_b, sem_sb)
            drain_strip(strip_a, sem_sa)
            process(r0, strip_a)

            @pl.when(r0 + 2 < NR)
            def _():
                fire_strip(r0 + 2, strip_a, sem_sa)

            for h in hb:
                h.wait()
            process(r0 + 1, strip_b)
            return carry

        lax.fori_loop(0, NR // 2, srbody, 0)

        if rem:
            slot = nblk - (nblk // RING) * RING
            pltpu.async_copy(
                ring.at[pl.ds(slot * CH, rem), :],
                out.at[b, pl.ds(nblk * CH, rem), :], sem_rem)
        for _ in range(nblk - max(0, nblk - (RING - 2))):
            pltpu.make_async_copy(
                out.at[0, pl.ds(0, CH), :], ring.at[pl.ds(0, CH), :],
                sem_wb).wait()
        if rem:
            pltpu.make_async_copy(
                out.at[0, pl.ds(0, rem), :], ring.at[pl.ds(0, rem), :],
                sem_rem).wait()

    return sc_kernel


def kernel(img, patch_indices):
    K = patch_indices.shape[0]
    Kp = max(K, PSZ)
    img2d = img.reshape(NB * CCH * H, W)
    q = (patch_indices % WW).astype(jnp.int32)
    qcol = q * PSZ
    if Kp != K:
        qcol = jnp.pad(qcol, (0, Kp - K))
    r = (patch_indices // WW).astype(jnp.int32)
    starts = jnp.sum(r[None, :] < jnp.arange(ROWS48, dtype=jnp.int32)[:, None],
                     axis=1, dtype=jnp.int32)
    return _make_sc_call(K, Kp)(img2d, qcol, starts)
